# baseline (speedup 1.0000x reference)
import functools
import numpy as np
import jax
import jax.numpy as jnp
from jax import lax
from jax.experimental import pallas as pl
from jax.experimental.pallas import tpu as pltpu
from jax.experimental.pallas import tpu_sc as plsc

_G, _C, _GD, _R = 2, 4, 512, 2
_L = 16
_TW = 128
_KH = _GD // 2
_NKC = _KH // _L
_JH = 4

_V0 = np.float32(np.arctanh(np.float64(0.5) / 2.002))
_V1 = np.float32(np.arctanh(np.float64(1.5) / 2.002))


def _steps(v):
    one = jnp.float32(1.0)
    zero = jnp.float32(0.0)
    s = jnp.where(v > _V0, one, zero)
    s = s + jnp.where(v >= _V1, one, zero)
    s = s + jnp.where(v >= -_V0, one, zero)
    s = s + jnp.where(v > -_V1, one, zero)
    return s


def _make_body(BG, T):
    TH = T // 2
    NW = TH // _TW

    def body(x_hbm, w_hbm, bias_hbm, out_hbm,
             wbuf, bbuf, xbuf, zbuf, obuf, xsem0, xsem1, osem0, osem1):
        cid = lax.axis_index("c")
        sid = lax.axis_index("s")
        wid = cid * 16 + sid
        bg = wid // 2
        th = wid % 2
        g = bg % _G
        tbase = th * TH

        pltpu.sync_copy(w_hbm.at[g], wbuf)
        pltpu.sync_copy(bias_hbm.at[g], bbuf)

        def xcopy(w, h, sem):
            t0 = tbase + w * _TW
            return pltpu.make_async_copy(
                x_hbm.at[bg, pl.ds(h * _KH, _KH), pl.ds(t0, _TW)],
                xbuf.at[h], sem)

        def ocopy(w, slot, sem):
            t0 = tbase + w * _TW
            return pltpu.make_async_copy(
                obuf.at[slot], out_hbm.at[bg, :, pl.ds(t0, _TW)], sem)

        xcopy(0, 0, xsem0).start()
        xcopy(0, 1, xsem1).start()

        def accum_half(h, lh, accs):
            half = jnp.uint32(0x8000)
            mask = jnp.uint32(0xFFFF0000)

            def kbody(kc, accs):
                accs = list(accs)
                k0 = kc * _L
                wvs = [wbuf[c, pl.ds(h * _KH + k0, _L)] for c in range(_C)]
                for kk in range(_L):
                    xvs = []
                    for j in range(_JH):
                        xv = xbuf[h, k0 + kk, pl.ds((lh * _JH + j) * _L, _L)]
                        u = lax.bitcast_convert_type(xv, jnp.uint32)
                        u = (u + half) & mask
                        xvs.append(lax.bitcast_convert_type(u, jnp.float32))
                    for c in range(_C):
                        w = wvs[c][kk]
                        for j in range(_JH):
                            accs[c * _JH + j] = accs[c * _JH + j] + xvs[j] * w
                return tuple(accs)

            return lax.fori_loop(0, _NKC, kbody, tuple(accs))

        def pair_body(jp, carry):
            for slot in range(2):
                w = jp * 2 + slot
                osem = osem0 if slot == 0 else osem1
                for h in range(2):
                    xsem = xsem0 if h == 0 else xsem1
                    xcopy(w, h, xsem).wait()
                    for lh in range(2):
                        if h == 0:
                            accs = [bbuf[c]
                                    for c in range(_C) for _ in range(_JH)]
                        else:
                            accs = [zbuf[c, pl.ds((lh * _JH + j) * _L, _L)]
                                    for c in range(_C) for j in range(_JH)]
                        accs = accum_half(h, lh, accs)
                        if h == 0:
                            for c in range(_C):
                                for j in range(_JH):
                                    zbuf[c,
                                         pl.ds((lh * _JH + j) * _L, _L)] = (
                                        accs[c * _JH + j])
                        else:
                            if lh == 0:
                                @pl.when(w >= 2)
                                def _():
                                    ocopy(0, slot, osem).wait()

                            for j in range(_JH):
                                idx0 = None
                                idx1 = None
                                for c in range(_C):
                                    z = accs[c * _JH + j]
                                    s0 = _steps(z)
                                    a1 = 4.0 * z - 2.0 * (s0 - 2.0)
                                    s1 = _steps(a1)
                                    w5 = jnp.float32(5.0 ** c)
                                    c0 = s0 * w5
                                    c1 = s1 * w5
                                    idx0 = c0 if idx0 is None else idx0 + c0
                                    idx1 = c1 if idx1 is None else idx1 + c1
                                lg = lh * _JH + j
                                obuf[slot, 0, pl.ds(lg * _L, _L)] = (
                                    idx0.astype(jnp.int32))
                                obuf[slot, 1, pl.ds(lg * _L, _L)] = (
                                    idx1.astype(jnp.int32))

                    @pl.when(w + 1 < NW)
                    def _():
                        xcopy(w + 1, h, xsem).start()

                ocopy(w, slot, osem).start()
            return carry

        lax.fori_loop(0, NW // 2, pair_body, 0)
        ocopy(0, 0, osem0).wait()
        ocopy(0, 1, osem1).wait()

    return body


@jax.jit
def kernel(x, Win, b_in):
    B, DIM, T = x.shape
    x2 = x.reshape(B * _G, _GD, T)
    wu = lax.bitcast_convert_type(Win, jnp.uint32)
    wu = (wu + jnp.uint32(0x7FFF) + ((wu >> 16) & jnp.uint32(1))) & jnp.uint32(
        0xFFFF0000)
    Wr = lax.bitcast_convert_type(wu, jnp.float32)
    bias_bc = jnp.broadcast_to(b_in[:, :, None], (_G, _C, _L))
    mesh = plsc.VectorSubcoreMesh(core_axis_name="c", subcore_axis_name="s")
    run = functools.partial(
        pl.kernel,
        mesh=mesh,
        out_type=jax.ShapeDtypeStruct((B * _G, _R, T), jnp.int32),
        scratch_types=[
            pltpu.VMEM((_C, _GD), jnp.float32),
            pltpu.VMEM((_C, _L), jnp.float32),
            pltpu.VMEM((2, _KH, _TW), jnp.float32),
            pltpu.VMEM((_C, _TW), jnp.float32),
            pltpu.VMEM((2, _R, _TW), jnp.int32),
            pltpu.SemaphoreType.DMA,
            pltpu.SemaphoreType.DMA,
            pltpu.SemaphoreType.DMA,
            pltpu.SemaphoreType.DMA,
        ],
    )(_make_body(B * _G, T))
    out = run(x2, Wr, bias_bc)
    return out.reshape(B, _G * _R, T)


# speedup vs baseline: 1.3497x; 1.3497x over previous
import functools
import numpy as np
import jax
import jax.numpy as jnp
from jax import lax
from jax.experimental import pallas as pl
from jax.experimental.pallas import tpu as pltpu
from jax.experimental.pallas import tpu_sc as plsc

_G, _C, _GD, _R = 2, 4, 512, 2
_L = 16
_TW = 128
_KH = _GD // 2
_KU = 8
_JH = 4

_V0 = np.float32(np.arctanh(np.float64(0.5) / 2.002))
_V1 = np.float32(np.arctanh(np.float64(1.5) / 2.002))


def _steps(v):
    one = jnp.float32(1.0)
    zero = jnp.float32(0.0)
    s = jnp.where(v > _V0, one, zero)
    s = s + jnp.where(v >= _V1, one, zero)
    s = s + jnp.where(v >= -_V0, one, zero)
    s = s + jnp.where(v > -_V1, one, zero)
    return s


def _rne_bf16(x):
    u = lax.bitcast_convert_type(x, jnp.uint32)
    u = (u + jnp.uint32(0x7FFF) + ((u >> 16) & jnp.uint32(1))) & jnp.uint32(
        0xFFFF0000)
    return lax.bitcast_convert_type(u, jnp.float32)


def _make_body(BG, T):
    TH = T // 2
    NW = TH // _TW

    def body(x_hbm, w_hbm, bias_hbm, out_hbm,
             wbuf, bbuf, xbuf, zbuf, obuf, xsem0, xsem1, osem0, osem1):
        cid = lax.axis_index("c")
        sid = lax.axis_index("s")
        wid = cid * 16 + sid
        bg = wid // 2
        th = wid % 2
        g = bg % _G
        tbase = th * TH

        pltpu.sync_copy(w_hbm.at[g], wbuf)
        pltpu.sync_copy(bias_hbm.at[g], bbuf)

        def xcopy(w, h, sem):
            t0 = tbase + w * _TW
            return pltpu.make_async_copy(
                x_hbm.at[bg, pl.ds(h * _KH, _KH), pl.ds(t0, _TW)],
                xbuf.at[h], sem)

        def ocopy(w, slot, sem):
            t0 = tbase + w * _TW
            return pltpu.make_async_copy(
                obuf.at[slot], out_hbm.at[bg, :, pl.ds(t0, _TW)], sem)

        xcopy(0, 0, xsem0).start()
        xcopy(0, 1, xsem1).start()

        def accum_half(h, lh, accs):
            def kbody(kb, accs):
                accs = list(accs)
                for kk in range(_KU):
                    k = kb * _KU + kk
                    wvs = [wbuf[c, pl.ds((h * _KH + k) * _L, _L)]
                           for c in range(_C)]
                    for j in range(_JH):
                        xv = _rne_bf16(
                            xbuf[h, k, pl.ds((lh * _JH + j) * _L, _L)])
                        for c in range(_C):
                            accs[c * _JH + j] = accs[c * _JH + j] + xv * wvs[c]
                return tuple(accs)

            return lax.fori_loop(0, _KH // _KU, kbody, tuple(accs))

        def pair_body(jp, carry):
            for slot in range(2):
                w = jp * 2 + slot
                osem = osem0 if slot == 0 else osem1
                for h in range(2):
                    xsem = xsem0 if h == 0 else xsem1
                    xcopy(w, h, xsem).wait()
                    for lh in range(2):
                        if h == 0:
                            accs = [bbuf[c]
                                    for c in range(_C) for _ in range(_JH)]
                        else:
                            accs = [zbuf[c, pl.ds((lh * _JH + j) * _L, _L)]
                                    for c in range(_C) for j in range(_JH)]
                        accs = accum_half(h, lh, accs)
                        if h == 0:
                            for c in range(_C):
                                for j in range(_JH):
                                    zbuf[c,
                                         pl.ds((lh * _JH + j) * _L, _L)] = (
                                        accs[c * _JH + j])
                        else:
                            if lh == 0:
                                @pl.when(w >= 2)
                                def _():
                                    ocopy(0, slot, osem).wait()

                            for j in range(_JH):
                                idx0 = None
                                idx1 = None
                                for c in range(_C):
                                    z = accs[c * _JH + j]
                                    s0 = _steps(z)
                                    a1 = 4.0 * z - 2.0 * (s0 - 2.0)
                                    s1 = _steps(a1)
                                    w5 = jnp.float32(5.0 ** c)
                                    c0 = s0 * w5
                                    c1 = s1 * w5
                                    idx0 = c0 if idx0 is None else idx0 + c0
                                    idx1 = c1 if idx1 is None else idx1 + c1
                                lg = lh * _JH + j
                                obuf[slot, 0, pl.ds(lg * _L, _L)] = (
                                    idx0.astype(jnp.int32))
                                obuf[slot, 1, pl.ds(lg * _L, _L)] = (
                                    idx1.astype(jnp.int32))

                    @pl.when(w + 1 < NW)
                    def _():
                        xcopy(w + 1, h, xsem).start()

                ocopy(w, slot, osem).start()
            return carry

        lax.fori_loop(0, NW // 2, pair_body, 0)
        ocopy(0, 0, osem0).wait()
        ocopy(0, 1, osem1).wait()

    return body


@jax.jit
def kernel(x, Win, b_in):
    B, DIM, T = x.shape
    x2 = x.reshape(B * _G, _GD, T)
    Wr = _rne_bf16(Win)
    Wbc = jnp.broadcast_to(
        Wr[:, :, :, None], (_G, _C, _GD, _L)).reshape(_G, _C, _GD * _L)
    bias_bc = jnp.broadcast_to(b_in[:, :, None], (_G, _C, _L))
    mesh = plsc.VectorSubcoreMesh(core_axis_name="c", subcore_axis_name="s")
    run = functools.partial(
        pl.kernel,
        mesh=mesh,
        out_type=jax.ShapeDtypeStruct((B * _G, _R, T), jnp.int32),
        scratch_types=[
            pltpu.VMEM((_C, _GD * _L), jnp.float32),
            pltpu.VMEM((_C, _L), jnp.float32),
            pltpu.VMEM((2, _KH, _TW), jnp.float32),
            pltpu.VMEM((_C, _TW), jnp.float32),
            pltpu.VMEM((2, _R, _TW), jnp.int32),
            pltpu.SemaphoreType.DMA,
            pltpu.SemaphoreType.DMA,
            pltpu.SemaphoreType.DMA,
            pltpu.SemaphoreType.DMA,
        ],
    )(_make_body(B * _G, T))
    out = run(x2, Wbc, bias_bc)
    return out.reshape(B, _G * _R, T)


# speedup vs baseline: 2.6097x; 1.9336x over previous
import functools
import numpy as np
import jax
import jax.numpy as jnp
from jax import lax
from jax.experimental import pallas as pl
from jax.experimental.pallas import tpu as pltpu
from jax.experimental.pallas import tpu_sc as plsc

_G, _C, _GD, _R = 2, 4, 512, 2
_L = 16
_TW = 128
_KH = _GD // 2
_KU = 4
_JH = 4

_V0 = np.float32(np.arctanh(np.float64(0.5) / 2.002))
_V1 = np.float32(np.arctanh(np.float64(1.5) / 2.002))


def _steps(v):
    one = jnp.float32(1.0)
    zero = jnp.float32(0.0)
    s = jnp.where(v > _V0, one, zero)
    s = s + jnp.where(v >= _V1, one, zero)
    s = s + jnp.where(v >= -_V0, one, zero)
    s = s + jnp.where(v > -_V1, one, zero)
    return s


def _rne_bf16(x):
    u = lax.bitcast_convert_type(x, jnp.uint32)
    u = (u + jnp.uint32(0x7FFF) + ((u >> 16) & jnp.uint32(1))) & jnp.uint32(
        0xFFFF0000)
    return lax.bitcast_convert_type(u, jnp.float32)


def _rhu_bf16(x):
    u = lax.bitcast_convert_type(x, jnp.uint32)
    u = (u + jnp.uint32(0x8000)) & jnp.uint32(0xFFFF0000)
    return lax.bitcast_convert_type(u, jnp.float32)


def _make_body(BG, T):
    TH = T // 2
    NW = TH // _TW

    def body(x_hbm, w_hbm, bias_hbm, out_hbm,
             wbuf, bbuf, xbuf, zbuf, obuf, xsem0, xsem1, osem0, osem1):
        cid = lax.axis_index("c")
        sid = lax.axis_index("s")
        wid = cid * 16 + sid
        bg = wid // 2
        th = wid % 2
        g = bg % _G
        tbase = th * TH

        pltpu.sync_copy(w_hbm.at[g], wbuf)
        pltpu.sync_copy(bias_hbm.at[g], bbuf)

        def xcopy(w, h, sem):
            t0 = tbase + w * _TW
            return pltpu.make_async_copy(
                x_hbm.at[bg, pl.ds(h * _KH, _KH), pl.ds(t0, _TW)],
                xbuf.at[h], sem)

        def ocopy(w, slot, sem):
            t0 = tbase + w * _TW
            return pltpu.make_async_copy(
                obuf.at[slot], out_hbm.at[bg, :, pl.ds(t0, _TW)], sem)

        xcopy(0, 0, xsem0).start()
        xcopy(0, 1, xsem1).start()

        def accum_half(h, lh, accs):
            def kbody(kb, accs):
                accs = list(accs)
                for kk in range(_KU):
                    k = kb * _KU + kk
                    wvs = [wbuf[c, pl.ds((h * _KH + k) * _L, _L)]
                           for c in range(_C)]
                    for j in range(_JH):
                        xv = _rhu_bf16(
                            xbuf[h, k, pl.ds((lh * _JH + j) * _L, _L)])
                        for c in range(_C):
                            accs[c * _JH + j] = accs[c * _JH + j] + xv * wvs[c]
                return tuple(accs)

            return lax.fori_loop(0, _KH // _KU, kbody, tuple(accs))

        def pair_body(jp, carry):
            for slot in range(2):
                w = jp * 2 + slot
                osem = osem0 if slot == 0 else osem1
                for h in range(2):
                    xsem = xsem0 if h == 0 else xsem1
                    xcopy(w, h, xsem).wait()
                    for lh in range(2):
                        if h == 0:
                            accs = [bbuf[c]
                                    for c in range(_C) for _ in range(_JH)]
                        else:
                            accs = [zbuf[c, pl.ds((lh * _JH + j) * _L, _L)]
                                    for c in range(_C) for j in range(_JH)]
                        accs = accum_half(h, lh, accs)
                        if h == 0:
                            for c in range(_C):
                                for j in range(_JH):
                                    zbuf[c,
                                         pl.ds((lh * _JH + j) * _L, _L)] = (
                                        accs[c * _JH + j])
                        else:
                            if lh == 0:
                                @pl.when(w >= 2)
                                def _():
                                    ocopy(0, slot, osem).wait()

                            for j in range(_JH):
                                idx0 = None
                                idx1 = None
                                for c in range(_C):
                                    z = accs[c * _JH + j]
                                    s0 = _steps(z)
                                    a1 = 4.0 * z - 2.0 * (s0 - 2.0)
                                    s1 = _steps(a1)
                                    w5 = jnp.float32(5.0 ** c)
                                    c0 = s0 * w5
                                    c1 = s1 * w5
                                    idx0 = c0 if idx0 is None else idx0 + c0
                                    idx1 = c1 if idx1 is None else idx1 + c1
                                lg = lh * _JH + j
                                obuf[slot, 0, pl.ds(lg * _L, _L)] = (
                                    idx0.astype(jnp.int32))
                                obuf[slot, 1, pl.ds(lg * _L, _L)] = (
                                    idx1.astype(jnp.int32))

                    @pl.when(w + 1 < NW)
                    def _():
                        xcopy(w + 1, h, xsem).start()

                ocopy(w, slot, osem).start()
            return carry

        lax.fori_loop(0, NW // 2, pair_body, 0)
        ocopy(0, 0, osem0).wait()
        ocopy(0, 1, osem1).wait()

    return body


@jax.jit
def kernel(x, Win, b_in):
    B, DIM, T = x.shape
    x2 = x.reshape(B * _G, _GD, T)
    Wr = _rne_bf16(Win)
    Wbc = jnp.broadcast_to(
        Wr[:, :, :, None], (_G, _C, _GD, _L)).reshape(_G, _C, _GD * _L)
    bias_bc = jnp.broadcast_to(b_in[:, :, None], (_G, _C, _L))
    mesh = plsc.VectorSubcoreMesh(core_axis_name="c", subcore_axis_name="s")
    run = functools.partial(
        pl.kernel,
        mesh=mesh,
        out_type=jax.ShapeDtypeStruct((B * _G, _R, T), jnp.int32),
        scratch_types=[
            pltpu.VMEM((_C, _GD * _L), jnp.float32),
            pltpu.VMEM((_C, _L), jnp.float32),
            pltpu.VMEM((2, _KH, _TW), jnp.float32),
            pltpu.VMEM((_C, _TW), jnp.float32),
            pltpu.VMEM((2, _R, _TW), jnp.int32),
            pltpu.SemaphoreType.DMA,
            pltpu.SemaphoreType.DMA,
            pltpu.SemaphoreType.DMA,
            pltpu.SemaphoreType.DMA,
        ],
    )(_make_body(B * _G, T))
    out = run(x2, Wbc, bias_bc)
    return out.reshape(B, _G * _R, T)


# speedup vs baseline: 2.6991x; 1.0343x over previous
import functools
import numpy as np
import jax
import jax.numpy as jnp
from jax import lax
from jax.experimental import pallas as pl
from jax.experimental.pallas import tpu as pltpu
from jax.experimental.pallas import tpu_sc as plsc

_G, _C, _GD, _R = 2, 4, 512, 2
_L = 16
_KC = 32
_KU = 4
_JH = 4

_V0 = np.float32(np.arctanh(np.float64(0.5) / 2.002))
_V1 = np.float32(np.arctanh(np.float64(1.5) / 2.002))


def _steps(v):
    one = jnp.float32(1.0)
    zero = jnp.float32(0.0)
    s = jnp.where(v > _V0, one, zero)
    s = s + jnp.where(v >= _V1, one, zero)
    s = s + jnp.where(v >= -_V0, one, zero)
    s = s + jnp.where(v > -_V1, one, zero)
    return s


def _rne_bf16(x):
    u = lax.bitcast_convert_type(x, jnp.uint32)
    u = (u + jnp.uint32(0x7FFF) + ((u >> 16) & jnp.uint32(1))) & jnp.uint32(
        0xFFFF0000)
    return lax.bitcast_convert_type(u, jnp.float32)


def _rhu_bf16(x):
    u = lax.bitcast_convert_type(x, jnp.uint32)
    u = (u + jnp.uint32(0x8000)) & jnp.uint32(0xFFFF0000)
    return lax.bitcast_convert_type(u, jnp.float32)


def _make_body(BG, T):
    TH = T // 2
    NLG = TH // (_JH * _L)
    NCH = _GD // _KC

    def body(x_hbm, w_hbm, bias_hbm, out_hbm,
             wbuf, bbuf, xbuf, zbuf, obuf, xsem0, xsem1):
        cid = lax.axis_index("c")
        sid = lax.axis_index("s")
        wid = cid * 16 + sid
        bg = wid // 2
        th = wid % 2
        g = bg % _G
        tbase = th * TH

        pltpu.sync_copy(w_hbm.at[g], wbuf)
        pltpu.sync_copy(bias_hbm.at[g], bbuf)

        def xcopy(ci, slot, sem):
            return pltpu.make_async_copy(
                x_hbm.at[bg, pl.ds(ci * _KC, _KC), pl.ds(tbase, TH)],
                xbuf.at[slot], sem)

        xcopy(0, 0, xsem0).start()
        xcopy(1, 1, xsem1).start()

        def init_body(i, carry):
            for c in range(_C):
                zbuf[c, pl.ds(i * _L, _L)] = bbuf[c]
            return carry

        lax.fori_loop(0, TH // _L, init_body, 0)

        def pair_body(jp, carry):
            for slot in range(2):
                ci = jp * 2 + slot
                xsem = xsem0 if slot == 0 else xsem1
                xcopy(ci, slot, xsem).wait()
                k0 = ci * _KC

                def lgb_body(lgb, carry2):
                    t0 = lgb * (_JH * _L)
                    accs = [zbuf[c, pl.ds(t0 + j * _L, _L)]
                            for c in range(_C) for j in range(_JH)]

                    def kbody(kb, accs):
                        accs = list(accs)
                        for kk in range(_KU):
                            k = kb * _KU + kk
                            wvs = [wbuf[c, pl.ds((k0 + k) * _L, _L)]
                                   for c in range(_C)]
                            for j in range(_JH):
                                xv = _rhu_bf16(
                                    xbuf[slot, k, pl.ds(t0 + j * _L, _L)])
                                for c in range(_C):
                                    accs[c * _JH + j] = (
                                        accs[c * _JH + j] + xv * wvs[c])
                        return tuple(accs)

                    accs = lax.fori_loop(0, _KC // _KU, kbody, tuple(accs))
                    for c in range(_C):
                        for j in range(_JH):
                            zbuf[c, pl.ds(t0 + j * _L, _L)] = (
                                accs[c * _JH + j])
                    return carry2

                lax.fori_loop(0, NLG, lgb_body, 0)

                @pl.when(ci + 2 < NCH)
                def _():
                    xcopy(ci + 2, slot, xsem).start()
            return carry

        lax.fori_loop(0, NCH // 2, pair_body, 0)

        def fsq_body(lgb, carry):
            t0 = lgb * (_JH * _L)
            for j in range(_JH):
                idx0 = None
                idx1 = None
                for c in range(_C):
                    z = zbuf[c, pl.ds(t0 + j * _L, _L)]
                    s0 = _steps(z)
                    a1 = 4.0 * z - 2.0 * (s0 - 2.0)
                    s1 = _steps(a1)
                    w5 = jnp.float32(5.0 ** c)
                    c0 = s0 * w5
                    c1 = s1 * w5
                    idx0 = c0 if idx0 is None else idx0 + c0
                    idx1 = c1 if idx1 is None else idx1 + c1
                obuf[0, pl.ds(t0 + j * _L, _L)] = idx0.astype(jnp.int32)
                obuf[1, pl.ds(t0 + j * _L, _L)] = idx1.astype(jnp.int32)
            return carry

        lax.fori_loop(0, NLG, fsq_body, 0)
        pltpu.sync_copy(obuf, out_hbm.at[bg, :, pl.ds(tbase, TH)])

    return body


@jax.jit
def kernel(x, Win, b_in):
    B, DIM, T = x.shape
    x2 = x.reshape(B * _G, _GD, T)
    Wr = _rne_bf16(Win)
    Wbc = jnp.broadcast_to(
        Wr[:, :, :, None], (_G, _C, _GD, _L)).reshape(_G, _C, _GD * _L)
    bias_bc = jnp.broadcast_to(b_in[:, :, None], (_G, _C, _L))
    TH = T // 2
    mesh = plsc.VectorSubcoreMesh(core_axis_name="c", subcore_axis_name="s")
    run = functools.partial(
        pl.kernel,
        mesh=mesh,
        out_type=jax.ShapeDtypeStruct((B * _G, _R, T), jnp.int32),
        scratch_types=[
            pltpu.VMEM((_C, _GD * _L), jnp.float32),
            pltpu.VMEM((_C, _L), jnp.float32),
            pltpu.VMEM((2, _KC, TH), jnp.float32),
            pltpu.VMEM((_C, TH), jnp.float32),
            pltpu.VMEM((_R, TH), jnp.int32),
            pltpu.SemaphoreType.DMA,
            pltpu.SemaphoreType.DMA,
        ],
    )(_make_body(B * _G, T))
    out = run(x2, Wbc, bias_bc)
    return out.reshape(B, _G * _R, T)


# speedup vs baseline: 2.9416x; 1.0898x over previous
import functools
import numpy as np
import jax
import jax.numpy as jnp
from jax import lax
from jax.experimental import pallas as pl
from jax.experimental.pallas import tpu as pltpu
from jax.experimental.pallas import tpu_sc as plsc

_G, _C, _GD, _R = 2, 4, 512, 2
_L = 16
_KC = 32
_KU = 4
_JH = 4
_TT = 512
_B_SC = 2
_HALF_L = 2.002
_BASIS = (1.0, 5.0, 25.0, 125.0)

_V0 = np.float32(np.arctanh(np.float64(0.5) / 2.002))
_V1 = np.float32(np.arctanh(np.float64(1.5) / 2.002))


def _tc_round(b):
    one = jnp.float32(1.0)
    zero = jnp.float32(0.0)
    q = jnp.where(b > 0.5, one, zero)
    q = q + jnp.where(b >= 1.5, one, zero)
    q = q - jnp.where(b < -0.5, one, zero)
    q = q - jnp.where(b <= -1.5, one, zero)
    return q


def _tc_body(x_ref, w_ref, b_ref, o_ref):
    xb = x_ref[0]
    for g in range(_G):
        xg = xb[g * _GD:(g + 1) * _GD, :]
        w = w_ref[g]
        z = jax.lax.dot_general(
            w, xg, (((1,), (0,)), ((), ())),
            preferred_element_type=jnp.float32)
        z = z + b_ref[g][:, None]
        q0 = _tc_round(jnp.tanh(z) * _HALF_L)
        q1 = _tc_round(jnp.tanh(4.0 * z - 2.0 * q0) * _HALF_L)
        idx0 = sum((q0[c] + 2.0) * _BASIS[c] for c in range(_C))
        idx1 = sum((q1[c] + 2.0) * _BASIS[c] for c in range(_C))
        o_ref[0, g * _R, :] = idx0.astype(jnp.int32)
        o_ref[0, g * _R + 1, :] = idx1.astype(jnp.int32)


def _tc_kernel(x, Win, b_in):
    B, DIM, T = x.shape
    grid = (B, T // _TT)
    return pl.pallas_call(
        _tc_body,
        grid=grid,
        in_specs=[
            pl.BlockSpec((1, DIM, _TT), lambda b, t: (b, 0, t)),
            pl.BlockSpec((_G, _C, _GD), lambda b, t: (0, 0, 0)),
            pl.BlockSpec((_G, _C), lambda b, t: (0, 0)),
        ],
        out_specs=pl.BlockSpec((1, _G * _R, _TT), lambda b, t: (b, 0, t)),
        out_shape=jax.ShapeDtypeStruct((B, _G * _R, T), jnp.int32),
    )(x, Win, b_in)


def _steps(v):
    one = jnp.float32(1.0)
    zero = jnp.float32(0.0)
    s = jnp.where(v > _V0, one, zero)
    s = s + jnp.where(v >= _V1, one, zero)
    s = s + jnp.where(v >= -_V0, one, zero)
    s = s + jnp.where(v > -_V1, one, zero)
    return s


def _rne_bf16(x):
    u = lax.bitcast_convert_type(x, jnp.uint32)
    u = (u + jnp.uint32(0x7FFF) + ((u >> 16) & jnp.uint32(1))) & jnp.uint32(
        0xFFFF0000)
    return lax.bitcast_convert_type(u, jnp.float32)


def _rhu_bf16(x):
    u = lax.bitcast_convert_type(x, jnp.uint32)
    u = (u + jnp.uint32(0x8000)) & jnp.uint32(0xFFFF0000)
    return lax.bitcast_convert_type(u, jnp.float32)


def _make_sc_body(BG, T):
    WPB = 32 // BG
    TH = T // WPB
    NLG = TH // (_JH * _L)
    NCH = _GD // _KC

    def body(x_hbm, w_hbm, bias_hbm, out_hbm,
             wbuf, bbuf, xbuf, zbuf, obuf, xsem0, xsem1):
        cid = lax.axis_index("c")
        sid = lax.axis_index("s")
        wid = cid * 16 + sid
        bg = wid // WPB
        th = wid % WPB
        g = bg % _G
        tbase = th * TH

        pltpu.sync_copy(w_hbm.at[g], wbuf)
        pltpu.sync_copy(bias_hbm.at[g], bbuf)

        def xcopy(ci, slot, sem):
            return pltpu.make_async_copy(
                x_hbm.at[bg, pl.ds(ci * _KC, _KC), pl.ds(tbase, TH)],
                xbuf.at[slot], sem)

        xcopy(0, 0, xsem0).start()
        xcopy(1, 1, xsem1).start()

        def init_body(i, carry):
            for c in range(_C):
                zbuf[c, pl.ds(i * _L, _L)] = bbuf[c]
            return carry

        lax.fori_loop(0, TH // _L, init_body, 0)

        def pair_body(jp, carry):
            for slot in range(2):
                ci = jp * 2 + slot
                xsem = xsem0 if slot == 0 else xsem1
                xcopy(ci, slot, xsem).wait()
                k0 = ci * _KC

                def lgb_body(lgb, carry2):
                    t0 = lgb * (_JH * _L)
                    accs = [zbuf[c, pl.ds(t0 + j * _L, _L)]
                            for c in range(_C) for j in range(_JH)]

                    def kbody(kb, accs):
                        accs = list(accs)
                        for kk in range(_KU):
                            k = kb * _KU + kk
                            wvs = [wbuf[c, pl.ds((k0 + k) * _L, _L)]
                                   for c in range(_C)]
                            for j in range(_JH):
                                xv = _rhu_bf16(
                                    xbuf[slot, k, pl.ds(t0 + j * _L, _L)])
                                for c in range(_C):
                                    accs[c * _JH + j] = (
                                        accs[c * _JH + j] + xv * wvs[c])
                        return tuple(accs)

                    accs = lax.fori_loop(0, _KC // _KU, kbody, tuple(accs))
                    for c in range(_C):
                        for j in range(_JH):
                            zbuf[c, pl.ds(t0 + j * _L, _L)] = (
                                accs[c * _JH + j])
                    return carry2

                lax.fori_loop(0, NLG, lgb_body, 0)

                @pl.when(ci + 2 < NCH)
                def _():
                    xcopy(ci + 2, slot, xsem).start()
            return carry

        lax.fori_loop(0, NCH // 2, pair_body, 0)

        def fsq_body(lgb, carry):
            t0 = lgb * (_JH * _L)
            for j in range(_JH):
                idx0 = None
                idx1 = None
                for c in range(_C):
                    z = zbuf[c, pl.ds(t0 + j * _L, _L)]
                    s0 = _steps(z)
                    a1 = 4.0 * z - 2.0 * (s0 - 2.0)
                    s1 = _steps(a1)
                    w5 = jnp.float32(5.0 ** c)
                    c0 = s0 * w5
                    c1 = s1 * w5
                    idx0 = c0 if idx0 is None else idx0 + c0
                    idx1 = c1 if idx1 is None else idx1 + c1
                obuf[0, pl.ds(t0 + j * _L, _L)] = idx0.astype(jnp.int32)
                obuf[1, pl.ds(t0 + j * _L, _L)] = idx1.astype(jnp.int32)
            return carry

        lax.fori_loop(0, NLG, fsq_body, 0)
        pltpu.sync_copy(obuf, out_hbm.at[bg, :, pl.ds(tbase, TH)])

    return body


def _sc_kernel(x, Win, b_in):
    B, DIM, T = x.shape
    x2 = x.reshape(B * _G, _GD, T)
    Wr = _rne_bf16(Win)
    Wbc = jnp.broadcast_to(
        Wr[:, :, :, None], (_G, _C, _GD, _L)).reshape(_G, _C, _GD * _L)
    bias_bc = jnp.broadcast_to(b_in[:, :, None], (_G, _C, _L))
    BG = B * _G
    TH = T // (32 // BG)
    mesh = plsc.VectorSubcoreMesh(core_axis_name="c", subcore_axis_name="s")
    run = functools.partial(
        pl.kernel,
        mesh=mesh,
        out_type=jax.ShapeDtypeStruct((BG, _R, T), jnp.int32),
        scratch_types=[
            pltpu.VMEM((_C, _GD * _L), jnp.float32),
            pltpu.VMEM((_C, _L), jnp.float32),
            pltpu.VMEM((2, _KC, TH), jnp.float32),
            pltpu.VMEM((_C, TH), jnp.float32),
            pltpu.VMEM((_R, TH), jnp.int32),
            pltpu.SemaphoreType.DMA,
            pltpu.SemaphoreType.DMA,
        ],
    )(_make_sc_body(BG, T))
    out = run(x2, Wbc, bias_bc)
    return out.reshape(B, _G * _R, T)


@jax.jit
def kernel(x, Win, b_in):
    B = x.shape[0]
    nb_tc = B - _B_SC
    out_tc = _tc_kernel(x[:nb_tc], Win, b_in)
    out_sc = _sc_kernel(x[nb_tc:], Win, b_in)
    return jnp.concatenate([out_tc, out_sc], axis=0)


# speedup vs baseline: 2.9422x; 1.0002x over previous
import functools
import numpy as np
import jax
import jax.numpy as jnp
from jax import lax
from jax.experimental import pallas as pl
from jax.experimental.pallas import tpu as pltpu
from jax.experimental.pallas import tpu_sc as plsc

_G, _C, _GD, _R = 2, 4, 512, 2
_L = 16
_KC = 32
_KU = 4
_JH = 4
_TT = 512
_B_SC = 2
_HALF_L = 2.002
_BASIS = (1.0, 5.0, 25.0, 125.0)

_V0 = np.float32(np.arctanh(np.float64(0.5) / 2.002))
_V1 = np.float32(np.arctanh(np.float64(1.5) / 2.002))


def _tc_round(b):
    one = jnp.float32(1.0)
    zero = jnp.float32(0.0)
    q = jnp.where(b > 0.5, one, zero)
    q = q + jnp.where(b >= 1.5, one, zero)
    q = q - jnp.where(b < -0.5, one, zero)
    q = q - jnp.where(b <= -1.5, one, zero)
    return q


def _tc_body(x_ref, w_ref, b_ref, o_ref):
    xb = x_ref[0]
    for g in range(_G):
        xg = xb[g * _GD:(g + 1) * _GD, :]
        w = w_ref[g]
        z = jax.lax.dot_general(
            w, xg, (((1,), (0,)), ((), ())),
            preferred_element_type=jnp.float32)
        z = z + b_ref[g][:, None]
        q0 = _tc_round(jnp.tanh(z) * _HALF_L)
        q1 = _tc_round(jnp.tanh(4.0 * z - 2.0 * q0) * _HALF_L)
        idx0 = sum((q0[c] + 2.0) * _BASIS[c] for c in range(_C))
        idx1 = sum((q1[c] + 2.0) * _BASIS[c] for c in range(_C))
        o_ref[0, g * _R, :] = idx0.astype(jnp.int32)
        o_ref[0, g * _R + 1, :] = idx1.astype(jnp.int32)


def _tc_kernel(x, Win, b_in):
    B, DIM, T = x.shape
    grid = (B, T // _TT)
    return pl.pallas_call(
        _tc_body,
        grid=grid,
        in_specs=[
            pl.BlockSpec((1, DIM, _TT), lambda b, t: (b, 0, t)),
            pl.BlockSpec((_G, _C, _GD), lambda b, t: (0, 0, 0)),
            pl.BlockSpec((_G, _C), lambda b, t: (0, 0)),
        ],
        out_specs=pl.BlockSpec((1, _G * _R, _TT), lambda b, t: (b, 0, t)),
        out_shape=jax.ShapeDtypeStruct((B, _G * _R, T), jnp.int32),
    )(x, Win, b_in)


def _steps(v):
    one = jnp.float32(1.0)
    zero = jnp.float32(0.0)
    s = jnp.where(v > _V0, one, zero)
    s = s + jnp.where(v >= _V1, one, zero)
    s = s + jnp.where(v >= -_V0, one, zero)
    s = s + jnp.where(v > -_V1, one, zero)
    return s


def _rne_bf16(x):
    u = lax.bitcast_convert_type(x, jnp.uint32)
    u = (u + jnp.uint32(0x7FFF) + ((u >> 16) & jnp.uint32(1))) & jnp.uint32(
        0xFFFF0000)
    return lax.bitcast_convert_type(u, jnp.float32)


def _rhu_bf16(x):
    u = lax.bitcast_convert_type(x, jnp.uint32)
    u = (u + jnp.uint32(0x8000)) & jnp.uint32(0xFFFF0000)
    return lax.bitcast_convert_type(u, jnp.float32)


def _make_sc_body(BG, T):
    WPB = 32 // BG
    TH = T // WPB
    NLG = TH // (_JH * _L)
    NCH = _GD // _KC

    def body(x_hbm, w_hbm, bias_hbm, out_hbm,
             wbuf, bbuf, xbuf, zbuf, obuf, xsem0, xsem1):
        cid = lax.axis_index("c")
        sid = lax.axis_index("s")
        wid = cid * 16 + sid
        bg = wid // WPB
        th = wid % WPB
        g = bg % _G
        tbase = th * TH

        pltpu.sync_copy(w_hbm.at[g], wbuf)
        pltpu.sync_copy(bias_hbm.at[g], bbuf)

        def xcopy(ci, slot, sem):
            return pltpu.make_async_copy(
                x_hbm.at[bg, pl.ds(ci * _KC, _KC), pl.ds(tbase, TH)],
                xbuf.at[slot], sem)

        xcopy(0, 0, xsem0).start()
        xcopy(1, 1, xsem1).start()

        def init_body(i, carry):
            for c in range(_C):
                zbuf[c, pl.ds(i * _L, _L)] = bbuf[c]
            return carry

        lax.fori_loop(0, TH // _L, init_body, 0)

        def pair_body(jp, carry):
            for slot in range(2):
                ci = jp * 2 + slot
                xsem = xsem0 if slot == 0 else xsem1
                xcopy(ci, slot, xsem).wait()
                k0 = ci * _KC

                def lgb_body(lgb, carry2):
                    t0 = lgb * (_JH * _L)
                    accs = [zbuf[c, pl.ds(t0 + j * _L, _L)]
                            for c in range(_C) for j in range(_JH)]

                    def kbody(kb, accs):
                        accs = list(accs)
                        for kk in range(_KU):
                            k = kb * _KU + kk
                            wvs = [wbuf[c, pl.ds((k0 + k) * _L, _L)]
                                   for c in range(_C)]
                            for j in range(_JH):
                                xv = _rhu_bf16(
                                    xbuf[slot, k, pl.ds(t0 + j * _L, _L)])
                                for c in range(_C):
                                    accs[c * _JH + j] = (
                                        accs[c * _JH + j] + xv * wvs[c])
                        return tuple(accs)

                    accs = lax.fori_loop(0, _KC // _KU, kbody, tuple(accs))
                    for c in range(_C):
                        for j in range(_JH):
                            zbuf[c, pl.ds(t0 + j * _L, _L)] = (
                                accs[c * _JH + j])
                    return carry2

                lax.fori_loop(0, NLG, lgb_body, 0)

                @pl.when(ci + 2 < NCH)
                def _():
                    xcopy(ci + 2, slot, xsem).start()
            return carry

        lax.fori_loop(0, NCH // 2, pair_body, 0)

        def fsq_body(lgb, carry):
            t0 = lgb * (_JH * _L)
            for j in range(_JH):
                idx0 = None
                idx1 = None
                for c in range(_C):
                    z = zbuf[c, pl.ds(t0 + j * _L, _L)]
                    s0 = _steps(z)
                    a1 = 4.0 * z - 2.0 * (s0 - 2.0)
                    s1 = _steps(a1)
                    w5 = jnp.float32(5.0 ** c)
                    c0 = s0 * w5
                    c1 = s1 * w5
                    idx0 = c0 if idx0 is None else idx0 + c0
                    idx1 = c1 if idx1 is None else idx1 + c1
                obuf[0, pl.ds(t0 + j * _L, _L)] = idx0.astype(jnp.int32)
                obuf[1, pl.ds(t0 + j * _L, _L)] = idx1.astype(jnp.int32)
            return carry

        lax.fori_loop(0, NLG, fsq_body, 0)
        pltpu.sync_copy(obuf, out_hbm.at[bg, :, pl.ds(tbase, TH)])

    return body


def _sc_kernel(x, Win, b_in):
    B, DIM, T = x.shape
    x2 = x.reshape(B * _G, _GD, T)
    Wr = _rne_bf16(Win)
    Wbc = jnp.broadcast_to(
        Wr[:, :, :, None], (_G, _C, _GD, _L)).reshape(_G, _C, _GD * _L)
    bias_bc = jnp.broadcast_to(b_in[:, :, None], (_G, _C, _L))
    BG = B * _G
    TH = T // (32 // BG)
    mesh = plsc.VectorSubcoreMesh(core_axis_name="c", subcore_axis_name="s")
    run = functools.partial(
        pl.kernel,
        mesh=mesh,
        out_type=jax.ShapeDtypeStruct((BG, _R, T), jnp.int32),
        scratch_types=[
            pltpu.VMEM((_C, _GD * _L), jnp.float32),
            pltpu.VMEM((_C, _L), jnp.float32),
            pltpu.VMEM((2, _KC, TH), jnp.float32),
            pltpu.VMEM((_C, TH), jnp.float32),
            pltpu.VMEM((_R, TH), jnp.int32),
            pltpu.SemaphoreType.DMA,
            pltpu.SemaphoreType.DMA,
        ],
    )(_make_sc_body(BG, T))
    out = run(x2, Wbc, bias_bc)
    return out.reshape(B, _G * _R, T)


@jax.jit
def kernel(x, Win, b_in):
    B = x.shape[0]
    nb_tc = B - _B_SC
    out_sc = _sc_kernel(x[nb_tc:], Win, b_in)
    out_tc = _tc_kernel(x[:nb_tc], Win, b_in)
    return jnp.concatenate([out_tc, out_sc], axis=0)
